# baseline (speedup 1.0000x reference)
import functools

import jax
import jax.numpy as jnp
from jax import lax
from jax.experimental import pallas as pl
from jax.experimental.pallas import tpu as pltpu
from jax.experimental.pallas import tpu_sc as plsc

_B = 16384
_D = 32


def _sc_gather_body(users_hbm, items_hbm, ut_hbm, it_hbm, u_out, i_out,
                    uidx_v, urows_v, iidx_v, irows_v, sem_u, sem_i,
                    *, nc, b_per_w):
    wid = lax.axis_index("s") * nc + lax.axis_index("c")
    base = wid * b_per_w
    pltpu.sync_copy(users_hbm.at[pl.ds(base, b_per_w)], uidx_v)
    pltpu.sync_copy(items_hbm.at[pl.ds(base, b_per_w)], iidx_v)
    cu = pltpu.async_copy(ut_hbm.at[uidx_v], urows_v, sem_u)
    ci = pltpu.async_copy(it_hbm.at[iidx_v], irows_v, sem_i)
    cu.wait()
    ci.wait()
    pltpu.sync_copy(urows_v, u_out.at[pl.ds(base, b_per_w)])
    pltpu.sync_copy(irows_v, i_out.at[pl.ds(base, b_per_w)])


@functools.cache
def _make_sc_gather():
    info = plsc.get_sparse_core_info()
    nc, ns = info.num_cores, info.num_subcores
    nw = nc * ns
    assert _B % (8 * nw) == 0
    b_per_w = _B // nw
    mesh = plsc.VectorSubcoreMesh(core_axis_name="c", subcore_axis_name="s")
    return pl.kernel(
        functools.partial(_sc_gather_body, nc=nc, b_per_w=b_per_w),
        out_type=[
            jax.ShapeDtypeStruct((_B, _D), jnp.float32),
            jax.ShapeDtypeStruct((_B, _D), jnp.float32),
        ],
        mesh=mesh,
        scratch_types=[
            pltpu.VMEM((b_per_w,), jnp.int32),
            pltpu.VMEM((b_per_w, _D), jnp.float32),
            pltpu.VMEM((b_per_w,), jnp.int32),
            pltpu.VMEM((b_per_w, _D), jnp.float32),
            pltpu.SemaphoreType.DMA,
            pltpu.SemaphoreType.DMA,
        ],
        compiler_params=pltpu.CompilerParams(use_tc_tiling_on_sc=False),
        name="sc_two_table_gather",
    )


def _tc_dense_body(u_ref, i_ref, wut_ref, bu_ref, wit_ref, bi_ref, o_ref):
    u = jnp.dot(u_ref[...], wut_ref[...], preferred_element_type=jnp.float32)
    v = jnp.dot(i_ref[...], wit_ref[...], preferred_element_type=jnp.float32)
    u = jnp.maximum(u + bu_ref[...], 0.0)
    v = jnp.maximum(v + bi_ref[...], 0.0)
    o_ref[...] = jnp.sum(u * v, axis=1, keepdims=True)


_G = 8
_CB = _B // _G


@functools.cache
def _make_tc_dense():
    return pl.pallas_call(
        _tc_dense_body,
        grid=(_G,),
        in_specs=[
            pl.BlockSpec((_CB, _D), lambda g: (g, 0)),
            pl.BlockSpec((_CB, _D), lambda g: (g, 0)),
            pl.BlockSpec((_D, _D), lambda g: (0, 0)),
            pl.BlockSpec((1, _D), lambda g: (0, 0)),
            pl.BlockSpec((_D, _D), lambda g: (0, 0)),
            pl.BlockSpec((1, _D), lambda g: (0, 0)),
        ],
        out_specs=pl.BlockSpec((_CB, 1), lambda g: (g, 0)),
        out_shape=jax.ShapeDtypeStruct((_B, 1), jnp.float32),
    )


def kernel(users, items, user_table, item_table, W_u, b_u, W_i, b_i):
    u_rows, i_rows = _make_sc_gather()(users, items, user_table, item_table)
    out = _make_tc_dense()(
        u_rows, i_rows,
        W_u.T, b_u.reshape(1, _D),
        W_i.T, b_i.reshape(1, _D),
    )
    return out.reshape(_B)


# speedup vs baseline: 1.4949x; 1.4949x over previous
import functools

import jax
import jax.numpy as jnp
from jax import lax
from jax.experimental import pallas as pl
from jax.experimental.pallas import tpu as pltpu
from jax.experimental.pallas import tpu_sc as plsc

_B = 16384
_D = 32
_UNROLL = 16


def _sc_gather_body(users_hbm, items_hbm, ut_hbm, it_hbm, u_out, i_out,
                    uidx_s, iidx_s, rows_v,
                    sem, *, nc, b_per_w):
    wid = lax.axis_index("s") * nc + lax.axis_index("c")
    base = wid * b_per_w

    pltpu.sync_copy(users_hbm.at[pl.ds(base, b_per_w)], uidx_s)
    pltpu.sync_copy(items_hbm.at[pl.ds(base, b_per_w)], iidx_s)

    def gather_one(idx_s, tbl, out):
        def body(step, carry):
            vec = idx_s[pl.ds(step * 16, 16)]
            for t in range(16):
                j = step * 16 + t
                r = vec[t]
                pltpu.make_async_copy(
                    tbl.at[pl.ds(r, 1)], rows_v.at[pl.ds(j, 1)], sem
                ).start()
            return carry
        lax.fori_loop(0, b_per_w // 16, body, 0, unroll=False)
        pltpu.make_async_copy(tbl.at[pl.ds(0, b_per_w)], rows_v, sem).wait()
        pltpu.sync_copy(rows_v, out.at[pl.ds(base, b_per_w)])

    gather_one(uidx_s, ut_hbm, u_out)
    gather_one(iidx_s, it_hbm, i_out)


@functools.cache
def _make_sc_gather():
    info = plsc.get_sparse_core_info()
    nc, ns = info.num_cores, info.num_subcores
    nw = nc * ns
    assert _B % (8 * nw) == 0
    b_per_w = _B // nw
    mesh = plsc.VectorSubcoreMesh(core_axis_name="c", subcore_axis_name="s")
    return pl.kernel(
        functools.partial(_sc_gather_body, nc=nc, b_per_w=b_per_w),
        out_type=[
            jax.ShapeDtypeStruct((_B, _D), jnp.float32),
            jax.ShapeDtypeStruct((_B, _D), jnp.float32),
        ],
        mesh=mesh,
        scratch_types=[
            pltpu.VMEM((b_per_w,), jnp.int32),
            pltpu.VMEM((b_per_w,), jnp.int32),
            pltpu.VMEM((b_per_w, _D), jnp.float32),
            pltpu.SemaphoreType.DMA,
        ],
        compiler_params=pltpu.CompilerParams(use_tc_tiling_on_sc=True),
        name="sc_two_table_gather",
    )


def _tc_dense_body(u_ref, i_ref, wut_ref, bu_ref, wit_ref, bi_ref, o_ref):
    u = jnp.dot(u_ref[...], wut_ref[...], preferred_element_type=jnp.float32)
    v = jnp.dot(i_ref[...], wit_ref[...], preferred_element_type=jnp.float32)
    u = jnp.maximum(u + bu_ref[...], 0.0)
    v = jnp.maximum(v + bi_ref[...], 0.0)
    o_ref[...] = jnp.sum(u * v, axis=1, keepdims=True)


_G = 8
_CB = _B // _G


@functools.cache
def _make_tc_dense():
    return pl.pallas_call(
        _tc_dense_body,
        grid=(_G,),
        in_specs=[
            pl.BlockSpec((_CB, _D), lambda g: (g, 0)),
            pl.BlockSpec((_CB, _D), lambda g: (g, 0)),
            pl.BlockSpec((_D, _D), lambda g: (0, 0)),
            pl.BlockSpec((1, _D), lambda g: (0, 0)),
            pl.BlockSpec((_D, _D), lambda g: (0, 0)),
            pl.BlockSpec((1, _D), lambda g: (0, 0)),
        ],
        out_specs=pl.BlockSpec((_CB, 1), lambda g: (g, 0)),
        out_shape=jax.ShapeDtypeStruct((_B, 1), jnp.float32),
    )


def kernel(users, items, user_table, item_table, W_u, b_u, W_i, b_i):
    u_rows, i_rows = _make_sc_gather()(users, items, user_table, item_table)
    out = _make_tc_dense()(
        u_rows, i_rows,
        W_u.T, b_u.reshape(1, _D),
        W_i.T, b_i.reshape(1, _D),
    )
    return out.reshape(_B)


# speedup vs baseline: 1.6368x; 1.0950x over previous
import functools

import jax
import jax.numpy as jnp
from jax import lax
from jax.experimental import pallas as pl
from jax.experimental.pallas import tpu as pltpu
from jax.experimental.pallas import tpu_sc as plsc

_B = 16384
_D = 32
_V = 1000000
_PACK = 4
_PC = 8192
_PSTEPS = (_V + _PC - 1) // _PC
_VP = _PSTEPS * (_PC // _PACK)


def _tc_pack_body(tT_ref, o_ref):
    for k in range(_PACK):
        sub = tT_ref[:, pl.ds(k * (_PC // _PACK), _PC // _PACK)]
        o_ref[:, k * _D:(k + 1) * _D] = sub.T


@functools.cache
def _make_tc_pack():
    return pl.pallas_call(
        _tc_pack_body,
        grid=(_PSTEPS,),
        in_specs=[pl.BlockSpec((_D, _PC), lambda g: (0, g))],
        out_specs=pl.BlockSpec((_PC // _PACK, _PACK * _D), lambda g: (g, 0)),
        out_shape=jax.ShapeDtypeStruct((_VP, _PACK * _D), jnp.float32),
    )


def _sc_gather_body(uq_hbm, iq_hbm, utp_hbm, itp_hbm, u_out, i_out,
                    uq_v, iq_v, grp_v, sem, *, nc, b_per_w):
    wid = lax.axis_index("s") * nc + lax.axis_index("c")
    base = wid * b_per_w

    pltpu.sync_copy(uq_hbm.at[pl.ds(base, b_per_w)], uq_v)
    pltpu.sync_copy(iq_hbm.at[pl.ds(base, b_per_w)], iq_v)

    pltpu.async_copy(utp_hbm.at[uq_v], grp_v, sem).wait()
    pltpu.sync_copy(grp_v, u_out.at[pl.ds(base, b_per_w)])
    pltpu.async_copy(itp_hbm.at[iq_v], grp_v, sem).wait()
    pltpu.sync_copy(grp_v, i_out.at[pl.ds(base, b_per_w)])


@functools.cache
def _make_sc_gather():
    info = plsc.get_sparse_core_info()
    nc, ns = info.num_cores, info.num_subcores
    nw = nc * ns
    assert _B % (8 * nw) == 0
    b_per_w = _B // nw
    mesh = plsc.VectorSubcoreMesh(core_axis_name="c", subcore_axis_name="s")
    return pl.kernel(
        functools.partial(_sc_gather_body, nc=nc, b_per_w=b_per_w),
        out_type=[
            jax.ShapeDtypeStruct((_B, _PACK * _D), jnp.float32),
            jax.ShapeDtypeStruct((_B, _PACK * _D), jnp.float32),
        ],
        mesh=mesh,
        scratch_types=[
            pltpu.VMEM((b_per_w,), jnp.int32),
            pltpu.VMEM((b_per_w,), jnp.int32),
            pltpu.VMEM((b_per_w, _PACK * _D), jnp.float32),
            pltpu.SemaphoreType.DMA,
        ],
        compiler_params=pltpu.CompilerParams(use_tc_tiling_on_sc=True),
        name="sc_two_table_gather",
    )


def _tc_dense_body(u4_ref, i4_ref, up_ref, ip_ref,
                   wut_ref, bu_ref, wit_ref, bi_ref, o_ref):
    def select(x4, phase):
        acc = jnp.zeros((x4.shape[0], _D), jnp.float32)
        for p in range(_PACK):
            m = (phase == p).astype(jnp.float32)
            acc = acc + x4[:, p * _D:(p + 1) * _D] * m
        return acc

    u = select(u4_ref[...], up_ref[...])
    v = select(i4_ref[...], ip_ref[...])
    u = jnp.dot(u, wut_ref[...], preferred_element_type=jnp.float32)
    v = jnp.dot(v, wit_ref[...], preferred_element_type=jnp.float32)
    u = jnp.maximum(u + bu_ref[...], 0.0)
    v = jnp.maximum(v + bi_ref[...], 0.0)
    o_ref[...] = jnp.sum(u * v, axis=1, keepdims=True)


_G = 8
_CB = _B // _G


@functools.cache
def _make_tc_dense():
    return pl.pallas_call(
        _tc_dense_body,
        grid=(_G,),
        in_specs=[
            pl.BlockSpec((_CB, _PACK * _D), lambda g: (g, 0)),
            pl.BlockSpec((_CB, _PACK * _D), lambda g: (g, 0)),
            pl.BlockSpec((_CB, 1), lambda g: (g, 0)),
            pl.BlockSpec((_CB, 1), lambda g: (g, 0)),
            pl.BlockSpec((_D, _D), lambda g: (0, 0)),
            pl.BlockSpec((1, _D), lambda g: (0, 0)),
            pl.BlockSpec((_D, _D), lambda g: (0, 0)),
            pl.BlockSpec((1, _D), lambda g: (0, 0)),
        ],
        out_specs=pl.BlockSpec((_CB, 1), lambda g: (g, 0)),
        out_shape=jax.ShapeDtypeStruct((_B, 1), jnp.float32),
    )


def kernel(users, items, user_table, item_table, W_u, b_u, W_i, b_i):
    utp = _make_tc_pack()(user_table.T)
    itp = _make_tc_pack()(item_table.T)
    sub = _PC // _PACK
    uq = (users // _PC) * sub + (users % sub)
    iq = (items // _PC) * sub + (items % sub)
    u4, i4 = _make_sc_gather()(uq, iq, utp, itp)
    up = ((users % _PC) // sub).reshape(_B, 1)
    ip = ((items % _PC) // sub).reshape(_B, 1)
    out = _make_tc_dense()(
        u4, i4, up, ip,
        W_u.T, b_u.reshape(1, _D),
        W_i.T, b_i.reshape(1, _D),
    )
    return out.reshape(_B)


# speedup vs baseline: 2.0827x; 1.2724x over previous
import functools

import jax
import jax.numpy as jnp
from jax import lax
from jax.experimental import pallas as pl
from jax.experimental.pallas import tpu as pltpu
from jax.experimental.pallas import tpu_sc as plsc

_B = 16384
_D = 32
_V = 1000000
_PACK = 4
_PC = 8192
_PSTEPS = (_V + _PC - 1) // _PC
_VP = _PSTEPS * (_PC // _PACK)


def _tc_pack_body(tT_ref, o_ref):
    n = _PC // _PACK
    col = lax.broadcasted_iota(jnp.int32, (_D, _PACK * _D), 1)
    row = lax.broadcasted_iota(jnp.int32, (_D, _PACK * _D), 0)
    acc = None
    for k in range(_PACK):
        ek = (col == row + k * _D).astype(jnp.float32)
        sub = tT_ref[:, pl.ds(k * n, n)]
        t = lax.dot_general(sub, ek, (((0,), (0,)), ((), ())),
                            preferred_element_type=jnp.float32)
        acc = t if acc is None else acc + t
    o_ref[...] = acc


@functools.cache
def _make_tc_pack():
    return pl.pallas_call(
        _tc_pack_body,
        grid=(_PSTEPS,),
        in_specs=[pl.BlockSpec((_D, _PC), lambda g: (0, g))],
        out_specs=pl.BlockSpec((_PC // _PACK, _PACK * _D), lambda g: (g, 0)),
        out_shape=jax.ShapeDtypeStruct((_VP, _PACK * _D), jnp.float32),
    )


def _sc_gather_body(q_hbm, tp_hbm, out, q_v, grp_v, sem, *, nc, b_per_w):
    wid = lax.axis_index("s") * nc + lax.axis_index("c")
    base = wid * b_per_w
    pltpu.sync_copy(q_hbm.at[pl.ds(base, b_per_w)], q_v)
    pltpu.async_copy(tp_hbm.at[q_v], grp_v, sem).wait()
    pltpu.sync_copy(grp_v, out.at[pl.ds(base, b_per_w)])


@functools.cache
def _make_sc_gather():
    info = plsc.get_sparse_core_info()
    nc, ns = info.num_cores, info.num_subcores
    nw = nc * ns
    assert _B % (8 * nw) == 0
    b_per_w = _B // nw
    mesh = plsc.VectorSubcoreMesh(core_axis_name="c", subcore_axis_name="s")
    return pl.kernel(
        functools.partial(_sc_gather_body, nc=nc, b_per_w=b_per_w),
        out_type=jax.ShapeDtypeStruct((_B, _PACK * _D), jnp.float32),
        mesh=mesh,
        scratch_types=[
            pltpu.VMEM((b_per_w,), jnp.int32),
            pltpu.VMEM((b_per_w, _PACK * _D), jnp.float32),
            pltpu.SemaphoreType.DMA,
        ],
        compiler_params=pltpu.CompilerParams(use_tc_tiling_on_sc=True),
        name="sc_packed_gather",
    )


def _tc_dense_body(u4_ref, i4_ref, up_ref, ip_ref,
                   wut_ref, bu_ref, wit_ref, bi_ref, o_ref):
    def select(x4, phase):
        acc = jnp.zeros((x4.shape[0], _D), jnp.float32)
        for p in range(_PACK):
            m = (phase == p).astype(jnp.float32)
            acc = acc + x4[:, p * _D:(p + 1) * _D] * m
        return acc

    u = select(u4_ref[...], up_ref[...])
    v = select(i4_ref[...], ip_ref[...])
    u = jnp.dot(u, wut_ref[...], preferred_element_type=jnp.float32)
    v = jnp.dot(v, wit_ref[...], preferred_element_type=jnp.float32)
    u = jnp.maximum(u + bu_ref[...], 0.0)
    v = jnp.maximum(v + bi_ref[...], 0.0)
    o_ref[...] = jnp.sum(u * v, axis=1, keepdims=True)


_G = 8
_CB = _B // _G


@functools.cache
def _make_tc_dense():
    return pl.pallas_call(
        _tc_dense_body,
        grid=(_G,),
        in_specs=[
            pl.BlockSpec((_CB, _PACK * _D), lambda g: (g, 0)),
            pl.BlockSpec((_CB, _PACK * _D), lambda g: (g, 0)),
            pl.BlockSpec((_CB, 1), lambda g: (g, 0)),
            pl.BlockSpec((_CB, 1), lambda g: (g, 0)),
            pl.BlockSpec((_D, _D), lambda g: (0, 0)),
            pl.BlockSpec((1, _D), lambda g: (0, 0)),
            pl.BlockSpec((_D, _D), lambda g: (0, 0)),
            pl.BlockSpec((1, _D), lambda g: (0, 0)),
        ],
        out_specs=pl.BlockSpec((_CB, 1), lambda g: (g, 0)),
        out_shape=jax.ShapeDtypeStruct((_B, 1), jnp.float32),
    )


def kernel(users, items, user_table, item_table, W_u, b_u, W_i, b_i):
    sub = _PC // _PACK
    uq = (users // _PC) * sub + (users % sub)
    iq = (items // _PC) * sub + (items % sub)
    utp = _make_tc_pack()(user_table.T)
    u4 = _make_sc_gather()(uq, utp)
    itp = _make_tc_pack()(item_table.T)
    i4 = _make_sc_gather()(iq, itp)
    up = ((users % _PC) // sub).reshape(_B, 1)
    ip = ((items % _PC) // sub).reshape(_B, 1)
    out = _make_tc_dense()(
        u4, i4, up, ip,
        W_u.T, b_u.reshape(1, _D),
        W_i.T, b_i.reshape(1, _D),
    )
    return out.reshape(_B)


# speedup vs baseline: 2.4780x; 1.1898x over previous
import functools

import jax
import jax.numpy as jnp
from jax import lax
from jax.experimental import pallas as pl
from jax.experimental.pallas import tpu as pltpu
from jax.experimental.pallas import tpu_sc as plsc

_B = 16384
_D = 32
_V = 1000000
_PACK = 4
_PC = 16384
_PSTEPS = (_V + _PC - 1) // _PC
_VP = _PSTEPS * (_PC // _PACK)


def _tc_pack_body(tT_ref, o_ref):
    n = _PC // _PACK
    col = lax.broadcasted_iota(jnp.int32, (_D, _PACK * _D), 1)
    row = lax.broadcasted_iota(jnp.int32, (_D, _PACK * _D), 0)
    acc = None
    for k in range(_PACK):
        ek = (col == row + k * _D).astype(jnp.float32)
        sub = tT_ref[:, pl.ds(k * n, n)]
        t = lax.dot_general(sub, ek, (((0,), (0,)), ((), ())),
                            preferred_element_type=jnp.float32)
        acc = t if acc is None else acc + t
    o_ref[...] = acc


@functools.cache
def _make_tc_pack():
    return pl.pallas_call(
        _tc_pack_body,
        grid=(_PSTEPS,),
        in_specs=[pl.BlockSpec((_D, _PC), lambda g: (0, g))],
        out_specs=pl.BlockSpec((_PC // _PACK, _PACK * _D), lambda g: (g, 0)),
        out_shape=jax.ShapeDtypeStruct((_VP, _PACK * _D), jnp.float32),
    )


def _sc_gather_body(q_hbm, tp_hbm, out, q_v, grp_v, sem, *, nc, b_per_w):
    wid = lax.axis_index("s") * nc + lax.axis_index("c")
    base = wid * b_per_w
    pltpu.sync_copy(q_hbm.at[pl.ds(base, b_per_w)], q_v)
    pltpu.async_copy(tp_hbm.at[q_v], grp_v, sem).wait()
    pltpu.sync_copy(grp_v, out.at[pl.ds(base, b_per_w)])


@functools.cache
def _make_sc_gather():
    info = plsc.get_sparse_core_info()
    nc, ns = info.num_cores, info.num_subcores
    nw = nc * ns
    assert _B % (8 * nw) == 0
    b_per_w = _B // nw
    mesh = plsc.VectorSubcoreMesh(core_axis_name="c", subcore_axis_name="s")
    return pl.kernel(
        functools.partial(_sc_gather_body, nc=nc, b_per_w=b_per_w),
        out_type=jax.ShapeDtypeStruct((_B, _PACK * _D), jnp.float32),
        mesh=mesh,
        scratch_types=[
            pltpu.VMEM((b_per_w,), jnp.int32),
            pltpu.VMEM((b_per_w, _PACK * _D), jnp.float32),
            pltpu.SemaphoreType.DMA,
        ],
        compiler_params=pltpu.CompilerParams(use_tc_tiling_on_sc=True),
        name="sc_packed_gather",
    )


def _tc_dense_body(u4_ref, i4_ref, up_ref, ip_ref,
                   wut_ref, bu_ref, wit_ref, bi_ref, o_ref):
    def select(x4, phase):
        acc = jnp.zeros((x4.shape[0], _D), jnp.float32)
        for p in range(_PACK):
            m = (phase == p).astype(jnp.float32)
            acc = acc + x4[:, p * _D:(p + 1) * _D] * m
        return acc

    u = select(u4_ref[...], up_ref[...])
    v = select(i4_ref[...], ip_ref[...])
    u = jnp.dot(u, wut_ref[...], preferred_element_type=jnp.float32)
    v = jnp.dot(v, wit_ref[...], preferred_element_type=jnp.float32)
    u = jnp.maximum(u + bu_ref[...], 0.0)
    v = jnp.maximum(v + bi_ref[...], 0.0)
    o_ref[...] = jnp.sum(u * v, axis=1, keepdims=True)


_G = 8
_CB = _B // _G


@functools.cache
def _make_tc_dense():
    return pl.pallas_call(
        _tc_dense_body,
        grid=(_G,),
        in_specs=[
            pl.BlockSpec((_CB, _PACK * _D), lambda g: (g, 0)),
            pl.BlockSpec((_CB, _PACK * _D), lambda g: (g, 0)),
            pl.BlockSpec((_CB, 1), lambda g: (g, 0)),
            pl.BlockSpec((_CB, 1), lambda g: (g, 0)),
            pl.BlockSpec((_D, _D), lambda g: (0, 0)),
            pl.BlockSpec((1, _D), lambda g: (0, 0)),
            pl.BlockSpec((_D, _D), lambda g: (0, 0)),
            pl.BlockSpec((1, _D), lambda g: (0, 0)),
        ],
        out_specs=pl.BlockSpec((_CB, 1), lambda g: (g, 0)),
        out_shape=jax.ShapeDtypeStruct((_B, 1), jnp.float32),
    )


def kernel(users, items, user_table, item_table, W_u, b_u, W_i, b_i):
    sub = _PC // _PACK
    uq = (users // _PC) * sub + (users % sub)
    iq = (items // _PC) * sub + (items % sub)
    utp = _make_tc_pack()(user_table.T)
    u4 = _make_sc_gather()(uq, utp)
    itp = _make_tc_pack()(item_table.T)
    i4 = _make_sc_gather()(iq, itp)
    up = ((users % _PC) // sub).reshape(_B, 1)
    ip = ((items % _PC) // sub).reshape(_B, 1)
    out = _make_tc_dense()(
        u4, i4, up, ip,
        W_u.T, b_u.reshape(1, _D),
        W_i.T, b_i.reshape(1, _D),
    )
    return out.reshape(_B)


# speedup vs baseline: 2.5812x; 1.0416x over previous
import functools

import jax
import jax.numpy as jnp
from jax import lax
from jax.experimental import pallas as pl
from jax.experimental.pallas import tpu as pltpu
from jax.experimental.pallas import tpu_sc as plsc

_B = 16384
_D = 32
_V = 1000000
_PACK = 4
_PC = 32768
_PSTEPS = (_V + _PC - 1) // _PC
_VP = _PSTEPS * (_PC // _PACK)


def _tc_pack_body(tT_ref, o_ref):
    n = _PC // _PACK
    col = lax.broadcasted_iota(jnp.int32, (_D, _PACK * _D), 1)
    row = lax.broadcasted_iota(jnp.int32, (_D, _PACK * _D), 0)
    acc = None
    for k in range(_PACK):
        ek = (col == row + k * _D).astype(jnp.float32)
        sub = tT_ref[:, pl.ds(k * n, n)]
        t = lax.dot_general(sub, ek, (((0,), (0,)), ((), ())),
                            preferred_element_type=jnp.float32)
        acc = t if acc is None else acc + t
    o_ref[...] = acc


@functools.cache
def _make_tc_pack():
    return pl.pallas_call(
        _tc_pack_body,
        grid=(_PSTEPS,),
        in_specs=[pl.BlockSpec((_D, _PC), lambda g: (0, g))],
        out_specs=pl.BlockSpec((_PC // _PACK, _PACK * _D), lambda g: (g, 0)),
        out_shape=jax.ShapeDtypeStruct((_VP, _PACK * _D), jnp.float32),
    )


def _sc_gather_body(q_hbm, tp_hbm, out, q_v, grp_v, sem, *, nc, b_per_w):
    wid = lax.axis_index("s") * nc + lax.axis_index("c")
    base = wid * b_per_w
    pltpu.sync_copy(q_hbm.at[pl.ds(base, b_per_w)], q_v)
    pltpu.async_copy(tp_hbm.at[q_v], grp_v, sem).wait()
    pltpu.sync_copy(grp_v, out.at[pl.ds(base, b_per_w)])


@functools.cache
def _make_sc_gather():
    info = plsc.get_sparse_core_info()
    nc, ns = info.num_cores, info.num_subcores
    nw = nc * ns
    assert _B % (8 * nw) == 0
    b_per_w = _B // nw
    mesh = plsc.VectorSubcoreMesh(core_axis_name="c", subcore_axis_name="s")
    return pl.kernel(
        functools.partial(_sc_gather_body, nc=nc, b_per_w=b_per_w),
        out_type=jax.ShapeDtypeStruct((_B, _PACK * _D), jnp.float32),
        mesh=mesh,
        scratch_types=[
            pltpu.VMEM((b_per_w,), jnp.int32),
            pltpu.VMEM((b_per_w, _PACK * _D), jnp.float32),
            pltpu.SemaphoreType.DMA,
        ],
        compiler_params=pltpu.CompilerParams(use_tc_tiling_on_sc=True),
        name="sc_packed_gather",
    )


def _tc_dense_body(u4_ref, i4_ref, up_ref, ip_ref,
                   wut_ref, bu_ref, wit_ref, bi_ref, o_ref):
    def select(x4, phase):
        acc = jnp.zeros((x4.shape[0], _D), jnp.float32)
        for p in range(_PACK):
            m = (phase == p).astype(jnp.float32)
            acc = acc + x4[:, p * _D:(p + 1) * _D] * m
        return acc

    u = select(u4_ref[...], up_ref[...])
    v = select(i4_ref[...], ip_ref[...])
    u = jnp.dot(u, wut_ref[...], preferred_element_type=jnp.float32)
    v = jnp.dot(v, wit_ref[...], preferred_element_type=jnp.float32)
    u = jnp.maximum(u + bu_ref[...], 0.0)
    v = jnp.maximum(v + bi_ref[...], 0.0)
    o_ref[...] = jnp.sum(u * v, axis=1, keepdims=True)


_G = 8
_CB = _B // _G


@functools.cache
def _make_tc_dense():
    return pl.pallas_call(
        _tc_dense_body,
        grid=(_G,),
        in_specs=[
            pl.BlockSpec((_CB, _PACK * _D), lambda g: (g, 0)),
            pl.BlockSpec((_CB, _PACK * _D), lambda g: (g, 0)),
            pl.BlockSpec((_CB, 1), lambda g: (g, 0)),
            pl.BlockSpec((_CB, 1), lambda g: (g, 0)),
            pl.BlockSpec((_D, _D), lambda g: (0, 0)),
            pl.BlockSpec((1, _D), lambda g: (0, 0)),
            pl.BlockSpec((_D, _D), lambda g: (0, 0)),
            pl.BlockSpec((1, _D), lambda g: (0, 0)),
        ],
        out_specs=pl.BlockSpec((_CB, 1), lambda g: (g, 0)),
        out_shape=jax.ShapeDtypeStruct((_B, 1), jnp.float32),
    )


def kernel(users, items, user_table, item_table, W_u, b_u, W_i, b_i):
    sub = _PC // _PACK
    uq = (users // _PC) * sub + (users % sub)
    iq = (items // _PC) * sub + (items % sub)
    utp = _make_tc_pack()(user_table.T)
    u4 = _make_sc_gather()(uq, utp)
    itp = _make_tc_pack()(item_table.T)
    i4 = _make_sc_gather()(iq, itp)
    up = ((users % _PC) // sub).reshape(_B, 1)
    ip = ((items % _PC) // sub).reshape(_B, 1)
    out = _make_tc_dense()(
        u4, i4, up, ip,
        W_u.T, b_u.reshape(1, _D),
        W_i.T, b_i.reshape(1, _D),
    )
    return out.reshape(_B)
